# baseline (speedup 1.0000x reference)
import functools

import jax
import jax.numpy as jnp
from jax.experimental import pallas as pl
from jax.experimental.pallas import tpu as pltpu


def _se_kernel(x_ref, w1_ref, b1_ref, w2_ref, b2_ref, o_ref, *, inv_hw):
    x = x_ref[...]
    pooled = jnp.sum(x, axis=0, dtype=jnp.float32) * inv_hw
    h = jnp.dot(pooled, w1_ref[...],
                preferred_element_type=jnp.float32) + b1_ref[...]
    h = jnp.maximum(h, 0.0)
    s = jnp.dot(h, w2_ref[...],
                preferred_element_type=jnp.float32) + b2_ref[...]
    s = jax.nn.sigmoid(s)
    o_ref[...] = x * s.astype(o_ref.dtype)[None, :, :]


def kernel(x, w1, b1, w2, b2):
    N, C, H, W = x.shape
    HW = H * W
    Cr = w1.shape[1]

    xt = jnp.transpose(x, (2, 3, 0, 1)).reshape(HW, N, C)
    w1f = w1.astype(jnp.float32)
    w2f = w2.astype(jnp.float32)
    b1r = b1.reshape(1, Cr).astype(jnp.float32)
    b2r = b2.reshape(1, C).astype(jnp.float32)

    bn = min(8, N)
    while N % bn:
        bn -= 1
    itemsize = x.dtype.itemsize
    cost = pl.CostEstimate(
        flops=2 * N * (2 * C * Cr) + 2 * N * C * HW,
        transcendentals=N * C,
        bytes_accessed=2 * N * C * HW * itemsize,
    )

    out = pl.pallas_call(
        functools.partial(_se_kernel, inv_hw=1.0 / HW),
        out_shape=jax.ShapeDtypeStruct((HW, N, C), x.dtype),
        grid=(N // bn,),
        in_specs=[
            pl.BlockSpec((HW, bn, C), lambda n: (0, n, 0)),
            pl.BlockSpec((C, Cr), lambda n: (0, 0)),
            pl.BlockSpec((1, Cr), lambda n: (0, 0)),
            pl.BlockSpec((Cr, C), lambda n: (0, 0)),
            pl.BlockSpec((1, C), lambda n: (0, 0)),
        ],
        out_specs=pl.BlockSpec((HW, bn, C), lambda n: (0, n, 0)),
        compiler_params=pltpu.CompilerParams(
            dimension_semantics=("parallel",),
            vmem_limit_bytes=64 * 1024 * 1024),
        cost_estimate=cost,
    )(xt, w1f, b1r, w2f, b2r)

    return out.reshape(H, W, N, C).transpose(2, 3, 0, 1)


# speedup vs baseline: 1.1674x; 1.1674x over previous
import functools

import jax
import jax.numpy as jnp
from jax.experimental import pallas as pl
from jax.experimental.pallas import tpu as pltpu


def _se_kernel(x_ref, w1_ref, b1_ref, w2_ref, b2_ref, o_ref, *, inv_hw):
    x = x_ref[...]
    pooled = jnp.sum(x, axis=0, dtype=jnp.float32) * inv_hw
    h = jnp.dot(pooled, w1_ref[...],
                preferred_element_type=jnp.float32) + b1_ref[...]
    h = jnp.maximum(h, 0.0)
    s = jnp.dot(h, w2_ref[...],
                preferred_element_type=jnp.float32) + b2_ref[...]
    s = jax.nn.sigmoid(s)
    o_ref[...] = x * s.astype(o_ref.dtype)[None, :, :]


def kernel(x, w1, b1, w2, b2):
    N, C, H, W = x.shape
    HW = H * W
    Cr = w1.shape[1]

    xt = jnp.transpose(x, (2, 3, 0, 1)).reshape(HW, N, C)
    w1f = w1.astype(jnp.float32)
    w2f = w2.astype(jnp.float32)
    b1r = b1.reshape(1, Cr).astype(jnp.float32)
    b2r = b2.reshape(1, C).astype(jnp.float32)

    bn = min(32, N)
    while N % bn:
        bn -= 1
    itemsize = x.dtype.itemsize
    cost = pl.CostEstimate(
        flops=2 * N * (2 * C * Cr) + 2 * N * C * HW,
        transcendentals=N * C,
        bytes_accessed=2 * N * C * HW * itemsize,
    )

    out = pl.pallas_call(
        functools.partial(_se_kernel, inv_hw=1.0 / HW),
        out_shape=jax.ShapeDtypeStruct((HW, N, C), x.dtype),
        grid=(N // bn,),
        in_specs=[
            pl.BlockSpec((HW, bn, C), lambda n: (0, n, 0)),
            pl.BlockSpec((C, Cr), lambda n: (0, 0)),
            pl.BlockSpec((1, Cr), lambda n: (0, 0)),
            pl.BlockSpec((Cr, C), lambda n: (0, 0)),
            pl.BlockSpec((1, C), lambda n: (0, 0)),
        ],
        out_specs=pl.BlockSpec((HW, bn, C), lambda n: (0, n, 0)),
        compiler_params=pltpu.CompilerParams(
            dimension_semantics=("parallel",),
            vmem_limit_bytes=64 * 1024 * 1024),
        cost_estimate=cost,
    )(xt, w1f, b1r, w2f, b2r)

    return out.reshape(H, W, N, C).transpose(2, 3, 0, 1)


# speedup vs baseline: 1.2278x; 1.0518x over previous
import functools

import jax
import jax.numpy as jnp
from jax.experimental import pallas as pl
from jax.experimental.pallas import tpu as pltpu


def _se_kernel(x_ref, w1t_ref, b1_ref, w2_ref, b2_ref, o_ref, *, inv_hw):
    x = x_ref[...]
    pooled = jnp.sum(x, axis=0, dtype=jnp.float32) * inv_hw
    h = jax.lax.dot_general(
        pooled, w1t_ref[...], (((1,), (1,)), ((), ())),
        preferred_element_type=jnp.float32) + b1_ref[...]
    h = jnp.maximum(h, 0.0)
    s = jnp.dot(h, w2_ref[...],
                preferred_element_type=jnp.float32) + b2_ref[...]
    s = jax.nn.sigmoid(s)
    o_ref[...] = x * s.astype(o_ref.dtype)[None, :, :]


def kernel(x, w1, b1, w2, b2):
    N, C, H, W = x.shape
    HW = H * W
    Cr = w1.shape[1]

    xt = jnp.transpose(x, (2, 3, 0, 1)).reshape(HW, N, C)
    w1t = w1.T.astype(jnp.float32)
    w2f = w2.astype(jnp.float32)
    b1r = b1.reshape(1, Cr).astype(jnp.float32)
    b2r = b2.reshape(1, C).astype(jnp.float32)

    bn = min(32, N)
    while N % bn:
        bn -= 1
    itemsize = x.dtype.itemsize
    cost = pl.CostEstimate(
        flops=2 * N * (2 * C * Cr) + 2 * N * C * HW,
        transcendentals=N * C,
        bytes_accessed=2 * N * C * HW * itemsize,
    )

    out = pl.pallas_call(
        functools.partial(_se_kernel, inv_hw=1.0 / HW),
        out_shape=jax.ShapeDtypeStruct((HW, N, C), x.dtype),
        grid=(N // bn,),
        in_specs=[
            pl.BlockSpec((HW, bn, C), lambda n: (0, n, 0)),
            pl.BlockSpec((Cr, C), lambda n: (0, 0)),
            pl.BlockSpec((1, Cr), lambda n: (0, 0)),
            pl.BlockSpec((Cr, C), lambda n: (0, 0)),
            pl.BlockSpec((1, C), lambda n: (0, 0)),
        ],
        out_specs=pl.BlockSpec((HW, bn, C), lambda n: (0, n, 0)),
        compiler_params=pltpu.CompilerParams(
            dimension_semantics=("parallel",),
            vmem_limit_bytes=64 * 1024 * 1024),
        cost_estimate=cost,
    )(xt, w1t, b1r, w2f, b2r)

    return out.reshape(H, W, N, C).transpose(2, 3, 0, 1)
